# baseline (speedup 1.0000x reference)
import functools

import jax
import jax.numpy as jnp
from jax import lax
from jax.experimental import pallas as pl
from jax.experimental.pallas import tpu as pltpu
from jax.experimental.pallas import tpu_sc as plsc

N = 10000
E = 320000
G = 64
D = 128

NC = 2
NS = 16
NW = NC * NS
CH = 128
NCHUNK = 80
SUP = 8
NSUP = NCHUNK // SUP
EPW = NCHUNK * CH
EPAD = NW * EPW - E
NA = N + 8
RPS = 624
TAIL = N - NS * RPS
TAILZ = NA - NS * RPS


def _sc_edge_agg_body(feats, ei, out, agg, rows0, rows1,
                      sbA, sbB, dbA, dbB, gs0, gs1, ss0, ss1,
                      xsA, xsB, dsA, dsB):
    c = lax.axis_index("c")
    s = lax.axis_index("s")
    wid = s * NC + c
    rows = (rows0, rows1)
    gs = (gs0, gs1)
    ss = (ss0, ss1)
    sb = (sbA, sbB)
    dbv = (dbA, dbB)
    xs = (xsA, xsB)
    ds = (dsA, dsB)

    def zrow(i, _):
        for j in range(D // 16):
            rows0[i, pl.ds(j * 16, 16)] = jnp.zeros((16,), jnp.float32)
        return _
    lax.fori_loop(0, CH, zrow, None)
    for r in range(RPS // CH):
        pltpu.sync_copy(rows0, agg.at[pl.ds(s * RPS + r * CH, CH)])
    pltpu.sync_copy(rows0.at[pl.ds(0, RPS % CH)],
                    agg.at[pl.ds(s * RPS + (RPS // CH) * CH, RPS % CH)])

    @pl.when(s == NS - 1)
    def _():
        pltpu.sync_copy(rows0.at[pl.ds(0, TAILZ)],
                        agg.at[pl.ds(NS * RPS, TAILZ)])

    plsc.subcore_barrier()

    def idx_load(part, sp, buf, sem):
        pltpu.async_copy(ei.at[part, wid, sp], buf, sem)

    def idx_wait(buf, sem):
        pltpu.make_async_copy(ei.at[0, wid, 0], buf, sem).wait()

    idx_load(0, 0, sbA, xsA)
    idx_load(1, 0, dbA, dsA)
    idx_load(0, 1, sbB, xsB)
    idx_load(1, 1, dbB, dsB)
    idx_wait(sbA, xsA)
    pltpu.async_copy(feats.at[sbA.at[0]], rows0, gs0)
    pltpu.async_copy(feats.at[sbA.at[1]], rows1, gs1)

    def super_step(i, _):
        for p in range(2):
            sp = 2 * i + p
            cur, nxt = p, 1 - p
            idx_wait(dbv[cur], ds[cur])
            for j in range(SUP):
                k = j % 2
                pltpu.make_async_copy(feats.at[sb[cur].at[j]], rows[k],
                                      gs[k]).wait()
                pltpu.async_copy(rows[k], agg.at[dbv[cur].at[j]], ss[k],
                                 add=True).wait()
                if j < SUP - 2:
                    pltpu.async_copy(feats.at[sb[cur].at[j + 2]], rows[k],
                                     gs[k])
                else:
                    @pl.when(sp + 1 < NSUP)
                    def _():
                        if j == SUP - 2:
                            idx_wait(sb[nxt], xs[nxt])
                        pltpu.async_copy(feats.at[sb[nxt].at[j - (SUP - 2)]],
                                         rows[k], gs[k])
                if j == SUP - 1:
                    @pl.when(sp + 2 < NSUP)
                    def _():
                        idx_load(0, sp + 2, sb[cur], xs[cur])
                        idx_load(1, sp + 2, dbv[cur], ds[cur])
        return _
    lax.fori_loop(0, NSUP // 2, super_step, None)

    plsc.subcore_barrier()
    pltpu.sync_copy(agg.at[pl.ds(s * RPS, RPS)], out.at[c, pl.ds(s * RPS, RPS)])

    @pl.when(s == NS - 1)
    def _():
        pltpu.sync_copy(agg.at[pl.ds(NS * RPS, TAIL)],
                        out.at[c, pl.ds(NS * RPS, TAIL)])


@functools.cache
def _sc_edge_agg():
    return pl.kernel(
        _sc_edge_agg_body,
        out_type=jax.ShapeDtypeStruct((NC, N, D), jnp.float32),
        mesh=plsc.VectorSubcoreMesh(
            core_axis_name="c", subcore_axis_name="s",
            num_cores=NC, num_subcores=NS,
        ),
        scratch_types=(
            [pltpu.VMEM_SHARED((NA, D), jnp.float32)]
            + [pltpu.VMEM((CH, D), jnp.float32)] * 2
            + [pltpu.VMEM((SUP, CH), jnp.int32)] * 4
            + [pltpu.SemaphoreType.DMA] * 8
        ),
    )


RB = 2000
NRB = N // RB


def _mlp_body(x_ref, p_ref, w1_ref, b1_ref, w2_ref, b2_ref, o_ref):
    h = x_ref[...] + p_ref[0] + p_ref[1]
    t = jnp.dot(h, w1_ref[...], preferred_element_type=jnp.float32) + b1_ref[...]
    t = jnp.maximum(t, 0.0)
    o = jnp.dot(t, w2_ref[...], preferred_element_type=jnp.float32) + b2_ref[...]
    o_ref[...] = jnp.maximum(o, 0.0)


_row_spec = pl.BlockSpec((RB, D), lambda i: (i, 0))
_parts_spec = pl.BlockSpec((NC, RB, D), lambda i: (0, i, 0))
_w_spec = pl.BlockSpec((D, D), lambda i: (0, 0))
_b_spec = pl.BlockSpec((1, D), lambda i: (0, 0))

_mlp = pl.pallas_call(
    _mlp_body,
    grid=(NRB,),
    in_specs=[_row_spec, _parts_spec, _w_spec, _b_spec, _w_spec, _b_spec],
    out_specs=_row_spec,
    out_shape=jax.ShapeDtypeStruct((N, D), jnp.float32),
)


def _log_softmax(z):
    m = jnp.max(z, axis=1, keepdims=True)
    e = jnp.exp(z - m)
    return z - m - jnp.log(jnp.sum(e, axis=1, keepdims=True))


def _mlp_pool_head_body(x_ref, p_ref, w1_ref, b1_ref, w2_ref, b2_ref,
                        batch_ref, code_ref, lin1W_ref, lin1b_ref, lin2W_ref,
                        lin2b_ref, fc1W_ref, fc1b_ref, fc2W_ref, fc2b_ref,
                        fc3W_ref, fc3b_ref, finA_ref, finB_ref, finb_ref,
                        o_ref, acc_ref):
    i = pl.program_id(0)
    h = x_ref[...] + p_ref[0] + p_ref[1]
    t = jnp.dot(h, w1_ref[...], preferred_element_type=jnp.float32) + b1_ref[...]
    t = jnp.maximum(t, 0.0)
    h2 = jnp.dot(t, w2_ref[...], preferred_element_type=jnp.float32) + b2_ref[...]
    h2 = jnp.maximum(h2, 0.0)
    b = batch_ref[0]
    gid = lax.broadcasted_iota(jnp.int32, (G, 1), 0)
    mask = (b == gid).astype(jnp.float32)
    part = jnp.dot(mask, h2, preferred_element_type=jnp.float32)

    @pl.when(i == 0)
    def _():
        acc_ref[...] = jnp.zeros_like(acc_ref)

    acc_ref[...] += part

    @pl.when(i == NRB - 1)
    def _():
        t1 = jnp.dot(acc_ref[...], lin1W_ref[...],
                     preferred_element_type=jnp.float32) + lin1b_ref[...]
        t1 = jnp.maximum(t1, 0.0)
        te = jnp.dot(t1, lin2W_ref[...],
                     preferred_element_type=jnp.float32) + lin2b_ref[...]
        c = jnp.dot(code_ref[...], fc1W_ref[...],
                    preferred_element_type=jnp.float32) + fc1b_ref[...]
        c = jnp.maximum(c, 0.0)
        c = jnp.dot(c, fc2W_ref[...], preferred_element_type=jnp.float32) + fc2b_ref[...]
        c = jnp.maximum(c, 0.0)
        z = jnp.dot(c, fc3W_ref[...], preferred_element_type=jnp.float32) + fc3b_ref[...]
        ce = _log_softmax(z)
        f = (jnp.dot(ce, finA_ref[...], preferred_element_type=jnp.float32)
             + jnp.dot(te, finB_ref[...], preferred_element_type=jnp.float32)
             + finb_ref[...])
        o_ref[...] = _log_softmax(f)


_full = lambda shape: pl.BlockSpec(shape, lambda i: tuple(0 for _ in shape))

_mlp_pool_head = pl.pallas_call(
    _mlp_pool_head_body,
    grid=(NRB,),
    in_specs=[
        _row_spec, _parts_spec, _w_spec, _b_spec, _w_spec, _b_spec,
        pl.BlockSpec((1, 1, RB), lambda i: (i, 0, 0)),
        _full((G, 256)), _full((D, D)), _b_spec, _full((D, D)), _b_spec,
        _full((256, D)), _b_spec, _full((D, D)), _b_spec, _full((D, D)),
        _b_spec, _full((D, G)), _full((D, G)), _full((1, G)),
    ],
    out_specs=pl.BlockSpec((G, G), lambda i: (0, 0)),
    out_shape=jax.ShapeDtypeStruct((G, G), jnp.float32),
    scratch_shapes=[pltpu.VMEM((G, D), jnp.float32)],
    compiler_params=pltpu.CompilerParams(dimension_semantics=("arbitrary",)),
)


def _fold_bn(W1, b1, g, beta, rm, rv):
    s = g / jnp.sqrt(rv + 1e-5)
    return W1 * s[None, :], (b1 - rm) * s + beta


def kernel(x, code_x, edge_index, batch, c1_W1, c1_b1, c1_g, c1_beta, c1_rm, c1_rv, c1_W2, c1_b2, c2_W1, c2_b1, c2_g, c2_beta, c2_rm, c2_rv, c2_W2, c2_b2, lin1_W, lin1_b, lin2_W, lin2_b, fc1_W, fc1_b, fc2_W, fc2_b, fc3_W, fc3_b, fin_W, fin_b):
    pad = jnp.stack([jnp.zeros((EPAD,), jnp.int32),
                     jnp.full((EPAD,), N, jnp.int32)])
    ei5 = jnp.concatenate([edge_index, pad], axis=1).reshape(
        2, NW, NSUP, SUP, CH)
    batch3 = batch.reshape(NRB, 1, RB)

    W1a, b1a = _fold_bn(c1_W1, c1_b1, c1_g, c1_beta, c1_rm, c1_rv)
    W1b, b1b = _fold_bn(c2_W1, c2_b1, c2_g, c2_beta, c2_rm, c2_rv)

    parts = _sc_edge_agg()(x, ei5)
    h1 = _mlp(x, parts, W1a, b1a.reshape(1, D), c1_W2, c1_b2.reshape(1, D))
    parts2 = _sc_edge_agg()(h1, ei5)
    return _mlp_pool_head(
        h1, parts2, W1b, b1b.reshape(1, D), c2_W2, c2_b2.reshape(1, D),
        batch3, code_x, lin1_W, lin1_b.reshape(1, D), lin2_W,
        lin2_b.reshape(1, D), fc1_W, fc1_b.reshape(1, D), fc2_W,
        fc2_b.reshape(1, D), fc3_W, fc3_b.reshape(1, D),
        fin_W[:D], fin_W[D:], fin_b.reshape(1, G))


# speedup vs baseline: 1.0415x; 1.0415x over previous
import functools

import jax
import jax.numpy as jnp
from jax import lax
from jax.experimental import pallas as pl
from jax.experimental.pallas import tpu as pltpu
from jax.experimental.pallas import tpu_sc as plsc

N = 10000
E = 320000
G = 64
D = 128

NC = 2
NS = 16
NW = NC * NS
CH = 128
NCHUNK = 80
SUP = 8
NSUP = NCHUNK // SUP
EPW = NCHUNK * CH
EPAD = NW * EPW - E
RPS = 624
TAIL = N - NS * RPS


def _sc_edge_agg_body(feats, ei, out, agg, rows0, rows1,
                      sbA, sbB, dbA, dbB, gs0, gs1, ss0, ss1,
                      xsA, xsB, dsA, dsB):
    c = lax.axis_index("c")
    s = lax.axis_index("s")
    wid = s * NC + c
    rows = (rows0, rows1)
    gs = (gs0, gs1)
    ss = (ss0, ss1)
    sb = (sbA, sbB)
    dbv = (dbA, dbB)
    xs = (xsA, xsB)
    ds = (dsA, dsB)

    def zrow(i, _):
        for j in range(D // 16):
            rows0[i, pl.ds(j * 16, 16)] = jnp.zeros((16,), jnp.float32)
        return _
    lax.fori_loop(0, CH, zrow, None)
    for r in range(RPS // CH):
        pltpu.sync_copy(rows0, agg.at[pl.ds(s * RPS + r * CH, CH)])
    pltpu.sync_copy(rows0.at[pl.ds(0, RPS % CH)],
                    agg.at[pl.ds(s * RPS + (RPS // CH) * CH, RPS % CH)])

    @pl.when(s == NS - 1)
    def _():
        pltpu.sync_copy(rows0.at[pl.ds(0, TAIL)], agg.at[pl.ds(NS * RPS, TAIL)])

    plsc.subcore_barrier()

    def idx_load(part, sp, buf, sem):
        pltpu.async_copy(ei.at[part, wid, sp], buf, sem)

    def idx_wait(buf, sem):
        pltpu.make_async_copy(ei.at[0, wid, 0], buf, sem).wait()

    idx_load(0, 0, sbA, xsA)
    idx_load(1, 0, dbA, dsA)
    idx_load(0, 1, sbB, xsB)
    idx_load(1, 1, dbB, dsB)
    idx_wait(sbA, xsA)
    pltpu.async_copy(feats.at[sbA.at[0]], rows0, gs0)
    pltpu.async_copy(feats.at[sbA.at[1]], rows1, gs1)

    def super_step(i, _):
        for p in range(2):
            sp = 2 * i + p
            cur, nxt = p, 1 - p
            idx_wait(dbv[cur], ds[cur])
            for j in range(SUP):
                k = j % 2
                pltpu.make_async_copy(feats.at[sb[cur].at[j]], rows[k],
                                      gs[k]).wait()
                pltpu.async_copy(rows[k], agg.at[dbv[cur].at[j]], ss[k],
                                 add=True).wait()
                if j < SUP - 2:
                    pltpu.async_copy(feats.at[sb[cur].at[j + 2]], rows[k],
                                     gs[k])
                else:
                    @pl.when(sp + 1 < NSUP)
                    def _():
                        if j == SUP - 2:
                            idx_wait(sb[nxt], xs[nxt])
                        pltpu.async_copy(feats.at[sb[nxt].at[j - (SUP - 2)]],
                                         rows[k], gs[k])
                if j == SUP - 1:
                    @pl.when(sp + 2 < NSUP)
                    def _():
                        idx_load(0, sp + 2, sb[cur], xs[cur])
                        idx_load(1, sp + 2, dbv[cur], ds[cur])
        return _
    lax.fori_loop(0, NSUP // 2, super_step, None)

    plsc.subcore_barrier()
    pltpu.sync_copy(agg.at[pl.ds(s * RPS, RPS)], out.at[c, pl.ds(s * RPS, RPS)])

    @pl.when(s == NS - 1)
    def _():
        pltpu.sync_copy(agg.at[pl.ds(NS * RPS, TAIL)],
                        out.at[c, pl.ds(NS * RPS, TAIL)])


@functools.cache
def _sc_edge_agg():
    return pl.kernel(
        _sc_edge_agg_body,
        out_type=jax.ShapeDtypeStruct((NC, N, D), jnp.float32),
        mesh=plsc.VectorSubcoreMesh(
            core_axis_name="c", subcore_axis_name="s",
            num_cores=NC, num_subcores=NS,
        ),
        scratch_types=(
            [pltpu.VMEM_SHARED((N, D), jnp.float32)]
            + [pltpu.VMEM((CH, D), jnp.float32)] * 2
            + [pltpu.VMEM((SUP, CH), jnp.int32)] * 4
            + [pltpu.SemaphoreType.DMA] * 8
        ),
    )


RB = 2000
NRB = N // RB


def _mlp_body(x_ref, p_ref, w1_ref, b1_ref, w2_ref, b2_ref, o_ref):
    h = x_ref[...] + p_ref[0] + p_ref[1]
    t = jnp.dot(h, w1_ref[...], preferred_element_type=jnp.float32) + b1_ref[...]
    t = jnp.maximum(t, 0.0)
    o = jnp.dot(t, w2_ref[...], preferred_element_type=jnp.float32) + b2_ref[...]
    o_ref[...] = jnp.maximum(o, 0.0)


_row_spec = pl.BlockSpec((RB, D), lambda i: (i, 0))
_parts_spec = pl.BlockSpec((NC, RB, D), lambda i: (0, i, 0))
_w_spec = pl.BlockSpec((D, D), lambda i: (0, 0))
_b_spec = pl.BlockSpec((1, D), lambda i: (0, 0))

_mlp = pl.pallas_call(
    _mlp_body,
    grid=(NRB,),
    in_specs=[_row_spec, _parts_spec, _w_spec, _b_spec, _w_spec, _b_spec],
    out_specs=_row_spec,
    out_shape=jax.ShapeDtypeStruct((N, D), jnp.float32),
)


def _log_softmax(z):
    m = jnp.max(z, axis=1, keepdims=True)
    e = jnp.exp(z - m)
    return z - m - jnp.log(jnp.sum(e, axis=1, keepdims=True))


def _mlp_pool_head_body(x_ref, p_ref, w1_ref, b1_ref, w2_ref, b2_ref,
                        batch_ref, code_ref, lin1W_ref, lin1b_ref, lin2W_ref,
                        lin2b_ref, fc1W_ref, fc1b_ref, fc2W_ref, fc2b_ref,
                        fc3W_ref, fc3b_ref, finA_ref, finB_ref, finb_ref,
                        o_ref, acc_ref):
    i = pl.program_id(0)
    h = x_ref[...] + p_ref[0] + p_ref[1]
    t = jnp.dot(h, w1_ref[...], preferred_element_type=jnp.float32) + b1_ref[...]
    t = jnp.maximum(t, 0.0)
    h2 = jnp.dot(t, w2_ref[...], preferred_element_type=jnp.float32) + b2_ref[...]
    h2 = jnp.maximum(h2, 0.0)
    b = batch_ref[0]
    gid = lax.broadcasted_iota(jnp.int32, (G, 1), 0)
    mask = (b == gid).astype(jnp.float32)
    part = jnp.dot(mask, h2, preferred_element_type=jnp.float32)

    @pl.when(i == 0)
    def _():
        acc_ref[...] = jnp.zeros_like(acc_ref)

    acc_ref[...] += part

    @pl.when(i == NRB - 1)
    def _():
        t1 = jnp.dot(acc_ref[...], lin1W_ref[...],
                     preferred_element_type=jnp.float32) + lin1b_ref[...]
        t1 = jnp.maximum(t1, 0.0)
        te = jnp.dot(t1, lin2W_ref[...],
                     preferred_element_type=jnp.float32) + lin2b_ref[...]
        c = jnp.dot(code_ref[...], fc1W_ref[...],
                    preferred_element_type=jnp.float32) + fc1b_ref[...]
        c = jnp.maximum(c, 0.0)
        c = jnp.dot(c, fc2W_ref[...], preferred_element_type=jnp.float32) + fc2b_ref[...]
        c = jnp.maximum(c, 0.0)
        z = jnp.dot(c, fc3W_ref[...], preferred_element_type=jnp.float32) + fc3b_ref[...]
        ce = _log_softmax(z)
        f = (jnp.dot(ce, finA_ref[...], preferred_element_type=jnp.float32)
             + jnp.dot(te, finB_ref[...], preferred_element_type=jnp.float32)
             + finb_ref[...])
        o_ref[...] = _log_softmax(f)


_full = lambda shape: pl.BlockSpec(shape, lambda i: tuple(0 for _ in shape))

_mlp_pool_head = pl.pallas_call(
    _mlp_pool_head_body,
    grid=(NRB,),
    in_specs=[
        _row_spec, _parts_spec, _w_spec, _b_spec, _w_spec, _b_spec,
        pl.BlockSpec((1, 1, RB), lambda i: (i, 0, 0)),
        _full((G, 256)), _full((D, D)), _b_spec, _full((D, D)), _b_spec,
        _full((256, D)), _b_spec, _full((D, D)), _b_spec, _full((D, D)),
        _b_spec, _full((D, G)), _full((D, G)), _full((1, G)),
    ],
    out_specs=pl.BlockSpec((G, G), lambda i: (0, 0)),
    out_shape=jax.ShapeDtypeStruct((G, G), jnp.float32),
    scratch_shapes=[pltpu.VMEM((G, D), jnp.float32)],
    compiler_params=pltpu.CompilerParams(dimension_semantics=("arbitrary",)),
)


def _fold_bn(W1, b1, g, beta, rm, rv):
    s = g / jnp.sqrt(rv + 1e-5)
    return W1 * s[None, :], (b1 - rm) * s + beta


def kernel(x, code_x, edge_index, batch, c1_W1, c1_b1, c1_g, c1_beta, c1_rm, c1_rv, c1_W2, c1_b2, c2_W1, c2_b1, c2_g, c2_beta, c2_rm, c2_rv, c2_W2, c2_b2, lin1_W, lin1_b, lin2_W, lin2_b, fc1_W, fc1_b, fc2_W, fc2_b, fc3_W, fc3_b, fin_W, fin_b):
    pad = jnp.stack([jnp.full((EPAD,), N, jnp.int32),
                     jnp.arange(EPAD, dtype=jnp.int32)])
    ei5 = jnp.concatenate([edge_index, pad], axis=1).reshape(
        2, NW, NSUP, SUP, CH)
    zrow = jnp.zeros((8, D), jnp.float32)
    xp = jnp.concatenate([x, zrow])
    batch3 = batch.reshape(NRB, 1, RB)

    W1a, b1a = _fold_bn(c1_W1, c1_b1, c1_g, c1_beta, c1_rm, c1_rv)
    W1b, b1b = _fold_bn(c2_W1, c2_b1, c2_g, c2_beta, c2_rm, c2_rv)

    parts = _sc_edge_agg()(xp, ei5)
    h1 = _mlp(x, parts, W1a, b1a.reshape(1, D), c1_W2, c1_b2.reshape(1, D))
    parts2 = _sc_edge_agg()(jnp.concatenate([h1, zrow]), ei5)
    return _mlp_pool_head(
        h1, parts2, W1b, b1b.reshape(1, D), c2_W2, c2_b2.reshape(1, D),
        batch3, code_x, lin1_W, lin1_b.reshape(1, D), lin2_W,
        lin2_b.reshape(1, D), fc1_W, fc1_b.reshape(1, D), fc2_W,
        fc2_b.reshape(1, D), fc3_W, fc3_b.reshape(1, D),
        fin_W[:D], fin_W[D:], fin_b.reshape(1, G))


# speedup vs baseline: 3.9208x; 3.7647x over previous
import functools

import jax
import jax.numpy as jnp
from jax import lax
from jax.experimental import pallas as pl
from jax.experimental.pallas import tpu as pltpu
from jax.experimental.pallas import tpu_sc as plsc

N = 10000
E = 320000
G = 64
D = 128

NC = 2
NS = 16
NW = NC * NS
CH = 128
NCHUNK = 80
SUP = 8
NSUP = NCHUNK // SUP
EPW = NCHUNK * CH
EPAD = NW * EPW - E
RPS = 624
TAIL = N - NS * RPS


def _sc_edge_agg_body(feats, ei, out, agg, rows0, rows1,
                      sbA, sbB, dbA, dbB, gs0, gs1, ss0, ss1,
                      xsA, xsB, dsA, dsB):
    c = lax.axis_index("c")
    s = lax.axis_index("s")
    wid = s * NC + c
    rows = (rows0, rows1)
    gs = (gs0, gs1)
    ss = (ss0, ss1)
    sb = (sbA, sbB)
    dbv = (dbA, dbB)
    xs = (xsA, xsB)
    ds = (dsA, dsB)

    def zrow(i, _):
        for j in range(D // 16):
            rows0[i, pl.ds(j * 16, 16)] = jnp.zeros((16,), jnp.float32)
        return _
    lax.fori_loop(0, CH, zrow, None)
    for r in range(RPS // CH):
        pltpu.sync_copy(rows0, agg.at[pl.ds(s * RPS + r * CH, CH)])
    pltpu.sync_copy(rows0.at[pl.ds(0, RPS % CH)],
                    agg.at[pl.ds(s * RPS + (RPS // CH) * CH, RPS % CH)])

    @pl.when(s == NS - 1)
    def _():
        pltpu.sync_copy(rows0.at[pl.ds(0, TAIL)], agg.at[pl.ds(NS * RPS, TAIL)])

    plsc.subcore_barrier()

    def idx_load(part, sp, buf, sem):
        pltpu.async_copy(ei.at[part, wid, sp], buf, sem)

    def idx_wait(buf, sem):
        pltpu.make_async_copy(ei.at[0, wid, 0], buf, sem).wait()

    idx_load(0, 0, sbA, xsA)
    idx_load(1, 0, dbA, dsA)
    idx_load(0, 1, sbB, xsB)
    idx_load(1, 1, dbB, dsB)
    idx_wait(sbA, xsA)
    pltpu.async_copy(feats.at[sbA.at[0]], rows0, gs0)
    pltpu.async_copy(feats.at[sbA.at[1]], rows1, gs1)

    def super_step(i, _):
        for p in range(2):
            sp = 2 * i + p
            cur, nxt = p, 1 - p
            idx_wait(dbv[cur], ds[cur])
            for j in range(SUP):
                k = j % 2
                pltpu.make_async_copy(feats.at[sb[cur].at[j]], rows[k],
                                      gs[k]).wait()
                pltpu.async_copy(rows[k], agg.at[dbv[cur].at[j]], ss[k],
                                 add=True).wait()
                if j < SUP - 2:
                    pltpu.async_copy(feats.at[sb[cur].at[j + 2]], rows[k],
                                     gs[k])
                else:
                    @pl.when(sp + 1 < NSUP)
                    def _():
                        if j == SUP - 2:
                            idx_wait(sb[nxt], xs[nxt])
                        pltpu.async_copy(feats.at[sb[nxt].at[j - (SUP - 2)]],
                                         rows[k], gs[k])
                if j == SUP - 1:
                    @pl.when(sp + 2 < NSUP)
                    def _():
                        idx_load(0, sp + 2, sb[cur], xs[cur])
                        idx_load(1, sp + 2, dbv[cur], ds[cur])
        return _
    lax.fori_loop(0, NSUP // 2, super_step, None)

    plsc.subcore_barrier()
    pltpu.sync_copy(agg.at[pl.ds(s * RPS, RPS)], out.at[c, pl.ds(s * RPS, RPS)])

    @pl.when(s == NS - 1)
    def _():
        pltpu.sync_copy(agg.at[pl.ds(NS * RPS, TAIL)],
                        out.at[c, pl.ds(NS * RPS, TAIL)])


@functools.cache
def _sc_edge_agg():
    return pl.kernel(
        _sc_edge_agg_body,
        out_type=jax.ShapeDtypeStruct((NC, N, D), jnp.float32),
        mesh=plsc.VectorSubcoreMesh(
            core_axis_name="c", subcore_axis_name="s",
            num_cores=NC, num_subcores=NS,
        ),
        scratch_types=(
            [pltpu.VMEM_SHARED((N, D), jnp.float32)]
            + [pltpu.VMEM((CH, D), jnp.float32)] * 2
            + [pltpu.VMEM((SUP, CH), jnp.int32)] * 4
            + [pltpu.SemaphoreType.DMA] * 8
        ),
    )


RB = 2000
NRB = N // RB


def _mlp_body(x_ref, p_ref, w1_ref, b1_ref, w2_ref, b2_ref, o_ref):
    h = x_ref[...] + p_ref[0] + p_ref[1]
    t = jnp.dot(h, w1_ref[...], preferred_element_type=jnp.float32) + b1_ref[...]
    t = jnp.maximum(t, 0.0)
    o = jnp.dot(t, w2_ref[...], preferred_element_type=jnp.float32) + b2_ref[...]
    o_ref[...] = jnp.maximum(o, 0.0)


_row_spec = pl.BlockSpec((RB, D), lambda i: (i, 0))
_parts_spec = pl.BlockSpec((NC, RB, D), lambda i: (0, i, 0))
_w_spec = pl.BlockSpec((D, D), lambda i: (0, 0))
_b_spec = pl.BlockSpec((1, D), lambda i: (0, 0))

_mlp = pl.pallas_call(
    _mlp_body,
    grid=(NRB,),
    in_specs=[_row_spec, _parts_spec, _w_spec, _b_spec, _w_spec, _b_spec],
    out_specs=_row_spec,
    out_shape=jax.ShapeDtypeStruct((N, D), jnp.float32),
)


def _log_softmax(z):
    m = jnp.max(z, axis=1, keepdims=True)
    e = jnp.exp(z - m)
    return z - m - jnp.log(jnp.sum(e, axis=1, keepdims=True))


def _mlp_pool_head_body(x_ref, p_ref, w1_ref, b1_ref, w2_ref, b2_ref,
                        batch_ref, code_ref, lin1W_ref, lin1b_ref, lin2W_ref,
                        lin2b_ref, fc1W_ref, fc1b_ref, fc2W_ref, fc2b_ref,
                        fc3W_ref, fc3b_ref, finA_ref, finB_ref, finb_ref,
                        o_ref, acc_ref):
    i = pl.program_id(0)
    h = x_ref[...] + p_ref[0] + p_ref[1]
    t = jnp.dot(h, w1_ref[...], preferred_element_type=jnp.float32) + b1_ref[...]
    t = jnp.maximum(t, 0.0)
    h2 = jnp.dot(t, w2_ref[...], preferred_element_type=jnp.float32) + b2_ref[...]
    h2 = jnp.maximum(h2, 0.0)
    b = batch_ref[0]
    gid = lax.broadcasted_iota(jnp.int32, (G, 1), 0)
    mask = (b == gid).astype(jnp.float32)
    part = jnp.dot(mask, h2, preferred_element_type=jnp.float32)

    @pl.when(i == 0)
    def _():
        acc_ref[...] = jnp.zeros_like(acc_ref)

    acc_ref[...] += part

    @pl.when(i == NRB - 1)
    def _():
        t1 = jnp.dot(acc_ref[...], lin1W_ref[...],
                     preferred_element_type=jnp.float32) + lin1b_ref[...]
        t1 = jnp.maximum(t1, 0.0)
        te = jnp.dot(t1, lin2W_ref[...],
                     preferred_element_type=jnp.float32) + lin2b_ref[...]
        c = jnp.dot(code_ref[...], fc1W_ref[...],
                    preferred_element_type=jnp.float32) + fc1b_ref[...]
        c = jnp.maximum(c, 0.0)
        c = jnp.dot(c, fc2W_ref[...], preferred_element_type=jnp.float32) + fc2b_ref[...]
        c = jnp.maximum(c, 0.0)
        z = jnp.dot(c, fc3W_ref[...], preferred_element_type=jnp.float32) + fc3b_ref[...]
        ce = _log_softmax(z)
        f = (jnp.dot(ce, finA_ref[...], preferred_element_type=jnp.float32)
             + jnp.dot(te, finB_ref[...], preferred_element_type=jnp.float32)
             + finb_ref[...])
        o_ref[...] = _log_softmax(f)


_full = lambda shape: pl.BlockSpec(shape, lambda i: tuple(0 for _ in shape))

_mlp_pool_head = pl.pallas_call(
    _mlp_pool_head_body,
    grid=(NRB,),
    in_specs=[
        _row_spec, _parts_spec, _w_spec, _b_spec, _w_spec, _b_spec,
        pl.BlockSpec((1, 1, RB), lambda i: (i, 0, 0)),
        _full((G, 256)), _full((D, D)), _b_spec, _full((D, D)), _b_spec,
        _full((256, D)), _b_spec, _full((D, D)), _b_spec, _full((D, D)),
        _b_spec, _full((D, G)), _full((D, G)), _full((1, G)),
    ],
    out_specs=pl.BlockSpec((G, G), lambda i: (0, 0)),
    out_shape=jax.ShapeDtypeStruct((G, G), jnp.float32),
    scratch_shapes=[pltpu.VMEM((G, D), jnp.float32)],
    compiler_params=pltpu.CompilerParams(dimension_semantics=("arbitrary",)),
)


def _fold_bn(W1, b1, g, beta, rm, rv):
    s = g / jnp.sqrt(rv + 1e-5)
    return W1 * s[None, :], (b1 - rm) * s + beta


def kernel(x, code_x, edge_index, batch, c1_W1, c1_b1, c1_g, c1_beta, c1_rm, c1_rv, c1_W2, c1_b2, c2_W1, c2_b1, c2_g, c2_beta, c2_rm, c2_rv, c2_W2, c2_b2, lin1_W, lin1_b, lin2_W, lin2_b, fc1_W, fc1_b, fc2_W, fc2_b, fc3_W, fc3_b, fin_W, fin_b):
    ar = jnp.arange(EPAD, dtype=jnp.int32)
    pad = jnp.stack([N + (ar % 128), ar])
    ei5 = jnp.concatenate([edge_index, pad], axis=1).reshape(
        2, NW, NSUP, SUP, CH)
    zrow = jnp.zeros((128, D), jnp.float32)
    xp = jnp.concatenate([x, zrow])
    batch3 = batch.reshape(NRB, 1, RB)

    W1a, b1a = _fold_bn(c1_W1, c1_b1, c1_g, c1_beta, c1_rm, c1_rv)
    W1b, b1b = _fold_bn(c2_W1, c2_b1, c2_g, c2_beta, c2_rm, c2_rv)

    parts = _sc_edge_agg()(xp, ei5)
    h1 = _mlp(x, parts, W1a, b1a.reshape(1, D), c1_W2, c1_b2.reshape(1, D))
    parts2 = _sc_edge_agg()(jnp.concatenate([h1, zrow]), ei5)
    return _mlp_pool_head(
        h1, parts2, W1b, b1b.reshape(1, D), c2_W2, c2_b2.reshape(1, D),
        batch3, code_x, lin1_W, lin1_b.reshape(1, D), lin2_W,
        lin2_b.reshape(1, D), fc1_W, fc1_b.reshape(1, D), fc2_W,
        fc2_b.reshape(1, D), fc3_W, fc3_b.reshape(1, D),
        fin_W[:D], fin_W[D:], fin_b.reshape(1, G))
